# Initial kernel scaffold; baseline (speedup 1.0000x reference)
import jax
import jax.numpy as jnp
from jax.experimental import pallas as pl


def kernel(params, elem_weights, elem_fea, self_fea_idx, nbr_fea_idx, cry_elem_idx):
    raise NotImplementedError("write your pallas kernel here")


# speedup vs baseline: 8.3645x; 8.3645x over previous
import functools

import jax
import jax.numpy as jnp
from jax import lax
from jax.experimental import pallas as pl
from jax.experimental.pallas import tpu as pltpu
from jax.experimental.pallas import tpu_sc as plsc

F32 = jnp.float32
I32 = jnp.int32

NC = 2
NS = 16
KCH = 128
WIN = 4096
ROWW = 256
TBLW = 128
TE = 1024
TN = 1024
HEADS = 3
FEA = 64
HID = 256


def _rup(x, m):
    return (x + m - 1) // m * m


def _embed_body(ef_ref, ew_ref, w_ref, b_ref, out_ref):
    x = ef_ref[...]
    t = jnp.dot(x, w_ref[...], preferred_element_type=F32) + b_ref[0:1, :]
    w = ew_ref[:, 0:1]
    out_ref[...] = jnp.concatenate(
        [t[:, : FEA - 1], w, w, jnp.zeros((x.shape[0], TBLW - FEA - 1), F32)],
        axis=1)


def _attn_core(x, logw, wh_ref, bh_ref, wg_ref, wm_ref, cst_ref, out_ref):
    h = jnp.dot(x, wh_ref[...], preferred_element_type=F32) + bh_ref[0:1, :]
    h = jnp.where(h > 0, h, 0.01 * h)
    msgs = []
    egs = []
    for hd in range(HEADS):
        hg = h[:, hd * 2 * HID: hd * 2 * HID + HID]
        hm = h[:, hd * 2 * HID + HID: (hd + 1) * 2 * HID]
        wg = wg_ref[0:1, hd * HID: (hd + 1) * HID]
        logit = jnp.sum(hg * wg, axis=1, keepdims=True) + cst_ref[1:2, hd:hd + 1]
        eg = jnp.exp(logit + logw * cst_ref[0:1, hd:hd + 1])
        msg = jnp.dot(hm, wm_ref[hd * HID: (hd + 1) * HID, :],
                      preferred_element_type=F32) + cst_ref[3 + hd:4 + hd, :FEA]
        msgs.append(msg * eg)
        egs.append(eg)
    out_ref[...] = jnp.concatenate(
        msgs + egs + [jnp.zeros((x.shape[0], ROWW - HEADS * (FEA + 1)), F32)],
        axis=1)


def _edge_body(sf_ref, nf_ref, wh_ref, bh_ref, wg_ref, wm_ref, cst_ref, out_ref):
    sf = sf_ref[...]
    nf = nf_ref[...]
    x = jnp.concatenate([sf[:, :FEA], nf[:, :FEA]], axis=1)
    logw = jnp.log(nf[:, FEA:FEA + 1])
    _attn_core(x, logw, wh_ref, bh_ref, wg_ref, wm_ref, cst_ref, out_ref)


def _cry_body(f_ref, wh_ref, bh_ref, wg_ref, wm_ref, cst_ref, out_ref):
    f = f_ref[...]
    x = f[:, :FEA]
    logw = jnp.log(f[:, FEA:FEA + 1])
    _attn_core(x, logw, wh_ref, bh_ref, wg_ref, wm_ref, cst_ref, out_ref)


def _head_mean(acc):
    s = jnp.zeros((acc.shape[0], FEA), F32)
    for hd in range(HEADS):
        s = s + acc[:, hd * FEA:(hd + 1) * FEA] / (
            acc[:, HEADS * FEA + hd:HEADS * FEA + hd + 1] + 1e-10)
    return s * (1.0 / HEADS)


def _combine_body(acc_ref, fea_ref, out_ref):
    f = fea_ref[...]
    new = f[:, :FEA] + _head_mean(acc_ref[...])
    out_ref[...] = jnp.concatenate(
        [new, f[:, FEA:FEA + 1], jnp.zeros((f.shape[0], TBLW - FEA - 1), F32)],
        axis=1)


def _final_body(acc_ref, out_ref):
    out_ref[...] = _head_mean(acc_ref[...])


def _make_gather(mp, tbl_rows):
    nchunks = mp // KCH // (NC * NS)
    mesh = plsc.VectorSubcoreMesh(core_axis_name="c", subcore_axis_name="s")

    @functools.partial(
        pl.kernel,
        out_type=(jax.ShapeDtypeStruct((mp, TBLW), F32),
                  jax.ShapeDtypeStruct((mp, TBLW), F32)),
        mesh=mesh,
        scratch_types=[
            pltpu.VMEM((KCH,), I32), pltpu.VMEM((KCH,), I32),
            pltpu.VMEM((KCH, TBLW), F32), pltpu.VMEM((KCH, TBLW), F32),
            pltpu.SemaphoreType.DMA, pltpu.SemaphoreType.DMA,
        ],
    )
    def gather_k(tbl, sidx, nidx, outs, outn, siv, niv, srow, nrow, sem_s, sem_n):
        wid = lax.axis_index("s") * NC + lax.axis_index("c")

        def body(i, carry):
            off = (wid * nchunks + i) * KCH
            pltpu.sync_copy(sidx.at[pl.ds(off, KCH)], siv)
            pltpu.sync_copy(nidx.at[pl.ds(off, KCH)], niv)
            cps = pltpu.async_copy(tbl.at[siv], srow, sem_s)
            cpn = pltpu.async_copy(tbl.at[niv], nrow, sem_n)
            cps.wait()
            cpn.wait()
            pltpu.sync_copy(srow, outs.at[pl.ds(off, KCH)])
            pltpu.sync_copy(nrow, outn.at[pl.ds(off, KCH)])
            return carry

        lax.fori_loop(0, nchunks, body, 0)

    return gather_k


def _make_scatter(mp, npx):
    nch_total = mp // KCH
    ngroups = (nch_total + NS - 1) // NS
    mesh = plsc.VectorSubcoreMesh(core_axis_name="c", subcore_axis_name="s")

    @functools.partial(
        pl.kernel,
        out_type=jax.ShapeDtypeStruct((npx, ROWW), F32),
        mesh=mesh,
        scratch_types=[
            pltpu.VMEM((KCH,), I32), pltpu.VMEM((KCH,), I32),
            pltpu.VMEM((KCH, ROWW), F32),
            pltpu.VMEM((16, ROWW), F32),
            pltpu.VMEM((16,), I32), pltpu.VMEM((16,), I32),
            pltpu.SemaphoreType.DMA, pltpu.SemaphoreType.DMA,
            pltpu.SemaphoreType.DMA, pltpu.SemaphoreType.DMA,
        ],
    )
    def scatter_k(vals, idx, tb, vb, out, idxv, relv, valv, zbuf, tbv, vbv,
                  sem_i, sem_v, sem_a, sem_d):
        cid = lax.axis_index("c")
        sid = lax.axis_index("s")
        wid = sid * NC + cid
        iota = lax.iota(I32, 16)

        def zrow(r, carry):
            for cc in range(ROWW // 16):
                zbuf[r, pl.ds(cc * 16, 16)] = jnp.zeros((16,), F32)
            return carry

        lax.fori_loop(0, 16, zrow, 0)
        pltpu.async_copy(tb.at[pl.ds(wid * 8, 16)], tbv, sem_d).wait()
        pltpu.async_copy(vb.at[pl.ds(wid * 8, 16)], vbv, sem_d).wait()
        tvec = tbv[pl.ds(0, 16)]
        vvec = vbv[pl.ds(0, 16)]
        e0 = tvec[0]
        e1 = tvec[8]
        v0 = vvec[0]
        v1 = vvec[8]

        def zero_body(i, carry):
            r = pl.multiple_of(v0 + i * 16, 16)

            @pl.when(r < v1)
            def _z():
                pltpu.async_copy(zbuf, out.at[pl.ds(r, 16)], sem_d).wait()

            return carry

        lax.fori_loop(0, npx // 16, zero_body, 0)

        c0 = e0 // KCH
        c1 = (e1 + KCH - 1) // KCH

        def group_body(g, carry):
            gbase = c0 + g * NS

            @pl.when(gbase < c1)
            def _grp():
                for k in range(NS):
                    cpos = gbase + k

                    @pl.when(cpos < c1)
                    def _chunk(cpos=cpos):
                        off = pl.multiple_of(cpos * KCH, KCH)
                        cp_i = pltpu.async_copy(idx.at[pl.ds(off, KCH)], idxv,
                                                sem_i)
                        cp_v = pltpu.async_copy(vals.at[pl.ds(off, KCH)], valv,
                                                sem_v)
                        cp_i.wait()
                        for j in range(KCH // 16):
                            iv = idxv[pl.ds(j * 16, 16)]
                            pos = iota + (off + j * 16)
                            good = (pos >= e0) & (pos < e1)
                            relv[pl.ds(j * 16, 16)] = jnp.where(good, iv, npx - 1)
                        cp_v.wait()
                        pltpu.async_copy(valv, out.at[relv], sem_a, add=True).wait()

            return carry

        lax.fori_loop(0, ngroups, group_body, 0)

    return scatter_k


def _attn_weights(heads, din):
    wh = jnp.concatenate(
        [jnp.concatenate([hp["gate"]["fc0"]["W"], hp["msg"]["fc0"]["W"]], axis=1)
         for hp in heads], axis=1)
    bh = jnp.concatenate(
        [jnp.concatenate([hp["gate"]["fc0"]["b"], hp["msg"]["fc0"]["b"]])
         for hp in heads])[None, :]
    bh = jnp.pad(bh, ((0, 7), (0, 0)))
    wg = jnp.concatenate([hp["gate"]["fc_out"]["W"][:, 0] for hp in heads])[None, :]
    wg = jnp.pad(wg, ((0, 7), (0, 0)))
    wm = jnp.concatenate([hp["msg"]["fc_out"]["W"] for hp in heads], axis=0)
    cst = jnp.zeros((8, 128), F32)
    for hd, hp in enumerate(heads):
        cst = cst.at[0, hd].set(hp["pow"][0])
        cst = cst.at[1, hd].set(hp["gate"]["fc_out"]["b"][0])
        cst = cst.at[3 + hd, :FEA].set(hp["msg"]["fc_out"]["b"])
    return wh, bh, wg, wm, cst


def kernel(params, elem_weights, elem_fea, self_fea_idx, nbr_fea_idx, cry_elem_idx):
    n, emb = elem_fea.shape
    m = self_fea_idx.shape[0]
    ncry = 10000

    npad_n = _rup(n, TN)
    mp = _rup(m, KCH * NC * NS)
    npx = npad_n + 16
    npc = _rup(ncry, 16) + 16

    pad_m = mp - m
    sidx_g = jnp.pad(self_fea_idx, (0, pad_m))
    nidx_g = jnp.pad(nbr_fea_idx, (0, pad_m))
    sidx_s = jnp.pad(self_fea_idx, (0, pad_m), constant_values=npx - 1)
    cidx_s = jnp.pad(cry_elem_idx, (0, npad_n - n), constant_values=npc - 1)

    def _partition(idx_sorted, rows, out_rows):
        probe = jnp.minimum(jnp.arange(NC * NS + 1) * (rows // (NC * NS)),
                            rows - 1)
        v = (idx_sorted[probe] // 16) * 16
        v = v.at[0].set(0).at[NC * NS].set(out_rows)
        t = jnp.searchsorted(idx_sorted, v).astype(I32)
        tp = jnp.zeros(((NC * NS + 1) * 8,), I32).at[
            jnp.arange(NC * NS + 1) * 8].set(t)
        vp = jnp.zeros(((NC * NS + 1) * 8,), I32).at[
            jnp.arange(NC * NS + 1) * 8].set(v.astype(I32))
        return tp, vp

    tb_n, vb_n = _partition(sidx_s, mp, npx)
    tb_c, vb_c = _partition(cidx_s, npad_n, npc)

    wep = jnp.pad(params["embed"]["W"], ((0, 0), (0, 1)))
    bep = jnp.pad(params["embed"]["b"], (0, 1))[None, :]
    bep = jnp.pad(bep, ((0, 7), (0, 0)))

    ef_pad = jnp.pad(elem_fea, ((0, npad_n - n), (0, 0)))
    ew_pad = jnp.pad(elem_weights, ((0, npad_n - n), (0, 0)), constant_values=1.0)
    ew_pad = jnp.pad(ew_pad, ((0, 0), (0, 7)))

    grid_n = npad_n // TN
    grid_e = mp // TE

    whole = lambda shape: pl.BlockSpec(shape, lambda i: (0, 0))
    rows = lambda t, w: pl.BlockSpec((t, w), lambda i: (i, 0))

    tbl = pl.pallas_call(
        _embed_body,
        grid=(grid_n,),
        in_specs=[rows(TN, emb), rows(TN, 8), whole((emb, FEA)), whole((8, FEA))],
        out_specs=rows(TN, TBLW),
        out_shape=jax.ShapeDtypeStruct((npad_n, TBLW), F32),
    )(ef_pad, ew_pad, wep, bep)

    gather_k = _make_gather(mp, npad_n)
    scat_n = _make_scatter(mp, npx)
    scat_c = _make_scatter(npad_n, npc)

    edge_call = pl.pallas_call(
        _edge_body,
        grid=(grid_e,),
        in_specs=[rows(TE, TBLW), rows(TE, TBLW), whole((2 * FEA, HEADS * 2 * HID)),
                  whole((8, HEADS * 2 * HID)), whole((8, HEADS * HID)),
                  whole((HEADS * HID, FEA)), whole((8, 128))],
        out_specs=rows(TE, ROWW),
        out_shape=jax.ShapeDtypeStruct((mp, ROWW), F32),
    )

    combine_call = pl.pallas_call(
        _combine_body,
        grid=(grid_n,),
        in_specs=[rows(TN, ROWW), rows(TN, TBLW)],
        out_specs=rows(TN, TBLW),
        out_shape=jax.ShapeDtypeStruct((npad_n, TBLW), F32),
    )

    for heads in params["graphs"]:
        wh, bh, wg, wm, cst = _attn_weights(heads, 2 * FEA)
        gs, gn = gather_k(tbl, sidx_g, nidx_g)
        vals = edge_call(gs, gn, wh, bh, wg, wm, cst)
        acc = scat_n(vals, sidx_s, tb_n, vb_n)
        tbl = combine_call(acc, tbl)

    whc, bhc, wgc, wmc, cstc = _attn_weights(params["cry"], FEA)
    cvals = pl.pallas_call(
        _cry_body,
        grid=(grid_n,),
        in_specs=[rows(TN, TBLW), whole((FEA, HEADS * 2 * HID)),
                  whole((8, HEADS * 2 * HID)), whole((8, HEADS * HID)),
                  whole((HEADS * HID, FEA)), whole((8, 128))],
        out_specs=rows(TN, ROWW),
        out_shape=jax.ShapeDtypeStruct((npad_n, ROWW), F32),
    )(tbl, whc, bhc, wgc, wmc, cstc)

    acc_c = scat_c(cvals, cidx_s, tb_c, vb_c)

    tc = 2000
    out = pl.pallas_call(
        _final_body,
        grid=(ncry // tc,),
        in_specs=[rows(tc, ROWW)],
        out_specs=rows(tc, FEA),
        out_shape=jax.ShapeDtypeStruct((ncry, FEA), F32),
    )(acc_c)
    return out
